# Initial kernel scaffold; baseline (speedup 1.0000x reference)
import jax
import jax.numpy as jnp
from jax.experimental import pallas as pl


def kernel(x, edge_index, batch, params):
    raise NotImplementedError("write your pallas kernel here")


# speedup vs baseline: 12.5747x; 12.5747x over previous
import functools

import jax
import jax.numpy as jnp
from jax.experimental import pallas as pl
from jax.experimental.pallas import tpu as pltpu

_N_GRAPHS = 64
_K_NN = 32
_EPS = 1e-5
_RT = 256
_CT = 256

_SMEM = getattr(pltpu, "SMEM", None)
if _SMEM is None:
    _SMEM = pltpu.TPUMemorySpace.SMEM


def _knn_kernel(bounds_ref, s_r_ref, s_c_ref, b_r_ref, b_c_ref, ft_ref,
                ft_t_ref, mean_ref, max_ref, *, s_dim, ct, k_nn):
    i = pl.program_id(0)
    c0 = bounds_ref[i, 0]
    c1 = bounds_ref[i, 1]
    sr = s_r_ref[...]
    br = b_r_ref[...][:, :1]
    rt = sr.shape[0]
    nf = ft_ref.shape[1]

    sr2 = sr * sr
    sq_r = sr2[:, 0:1]
    for dd in range(1, s_dim):
        sq_r = sq_r + sr2[:, dd:dd + 1]

    def d2_same(c):
        sc = s_c_ref[:, pl.ds(c * ct, ct)]
        dot = jax.lax.dot(sr, sc, preferred_element_type=jnp.float32)
        sc2 = sc * sc
        sq_c = sc2[0:1, :]
        for dd in range(1, s_dim):
            sq_c = sq_c + sc2[dd:dd + 1, :]
        d2 = sq_r + sq_c - 2.0 * dot
        bc = b_c_ref[:1, pl.ds(c * ct, ct)]
        return d2, br == bc

    def k_step(_, t):
        def c_step(c, acc):
            d, same = d2_same(c)
            cand = jnp.where(same & (d > t), d, jnp.inf)
            return jnp.minimum(acc, jnp.min(cand, axis=1, keepdims=True))
        return jax.lax.fori_loop(c0, c1, c_step,
                                 jnp.full((rt, 1), jnp.inf, jnp.float32))

    t = jax.lax.fori_loop(0, k_nn, k_step,
                          jnp.full((rt, 1), -jnp.inf, jnp.float32))

    def agg_step(c, carry):
        msum, mmax, cnt = carry
        d2, same = d2_same(c)
        sel = same & (d2 <= t)
        sc = s_c_ref[:, pl.ds(c * ct, ct)]
        d = jnp.zeros((rt, ct), jnp.float32)
        for dd in range(s_dim):
            diff = sr[:, dd:dd + 1] - sc[dd:dd + 1, :]
            d = d + diff * diff
        w = jnp.where(sel, jnp.exp(-10.0 * d), 0.0)
        ftc = ft_ref[pl.ds(c * ct, ct), :]
        msum = msum + jax.lax.dot(w, ftc, preferred_element_type=jnp.float32)
        cnt = cnt + jnp.sum(sel.astype(jnp.float32), axis=1, keepdims=True)
        nf = ft_ref.shape[1]
        cols = []
        for d in range(nf):
            fr = ft_t_ref[d:d + 1, pl.ds(c * ct, ct)]
            m2 = jnp.where(sel, w * fr, -jnp.inf)
            cols.append(jnp.max(m2, axis=1, keepdims=True))
        mmax = jnp.maximum(mmax, jnp.concatenate(cols, axis=1))
        return msum, mmax, cnt

    init = (jnp.zeros((rt, nf), jnp.float32),
            jnp.full((rt, nf), -jnp.inf, jnp.float32),
            jnp.zeros((rt, 1), jnp.float32))
    msum, mmax, cnt = jax.lax.fori_loop(c0, c1, agg_step, init)
    mean_ref[...] = msum / jnp.maximum(cnt, 1.0)
    max_ref[...] = jnp.where(jnp.isfinite(mmax), mmax, 0.0)


def _knn_aggregate(s, feat, batch, n_graphs, k_nn, interpret=False):
    n, s_dim = s.shape
    nf = feat.shape[1]
    npad = ((n + _RT - 1) // _RT) * _RT
    nt = npad // _RT
    pad = npad - n

    s_r = jnp.pad(s, ((0, pad), (0, 8 - s_dim)))
    s_c = s_r.T.reshape(8, npad)
    b_r = jnp.pad(batch.astype(jnp.float32), (0, pad),
                  constant_values=float(n_graphs + 63))
    b_c = jnp.pad(batch.astype(jnp.float32), (0, pad),
                  constant_values=float(n_graphs + 191))
    b_r = jnp.broadcast_to(b_r[:, None], (npad, 8))
    b_c = jnp.broadcast_to(b_c[None, :], (8, npad))
    ft = jnp.pad(feat, ((0, pad), (0, 0)))
    ft_t = ft.T.reshape(nf, npad)

    gids = jnp.arange(n_graphs, dtype=batch.dtype)
    g_start = jnp.searchsorted(batch, gids, side="left").astype(jnp.int32)
    g_end = jnp.searchsorted(batch, gids, side="right").astype(jnp.int32)
    first = jnp.minimum(jnp.arange(nt) * _RT, n - 1)
    last = jnp.minimum(jnp.arange(nt) * _RT + _RT - 1, n - 1)
    lo = g_start[batch[first]]
    hi = g_end[batch[last]]
    bounds = jnp.stack([lo // _CT, (hi + _CT - 1) // _CT], axis=1)

    kern = functools.partial(_knn_kernel, s_dim=s_dim, ct=_CT, k_nn=k_nn)
    mean_a, max_a = pl.pallas_call(
        kern,
        grid=(nt,),
        in_specs=[
            pl.BlockSpec(memory_space=_SMEM),
            pl.BlockSpec((_RT, 8), lambda i: (i, 0)),
            pl.BlockSpec((8, npad), lambda i: (0, 0)),
            pl.BlockSpec((_RT, 8), lambda i: (i, 0)),
            pl.BlockSpec((8, npad), lambda i: (0, 0)),
            pl.BlockSpec((npad, nf), lambda i: (0, 0)),
            pl.BlockSpec((nf, npad), lambda i: (0, 0)),
        ],
        out_specs=[
            pl.BlockSpec((_RT, nf), lambda i: (i, 0)),
            pl.BlockSpec((_RT, nf), lambda i: (i, 0)),
        ],
        out_shape=[
            jax.ShapeDtypeStruct((npad, nf), jnp.float32),
            jax.ShapeDtypeStruct((npad, nf), jnp.float32),
        ],
        compiler_params=pltpu.CompilerParams(
            dimension_semantics=("arbitrary",)),
        interpret=interpret,
    )(bounds, s_r, s_c, b_r, b_c, ft, ft_t)
    return mean_a[:n], max_a[:n]


def _matmul_kernel(x_ref, w_ref, b_ref, o_ref):
    o_ref[...] = (jax.lax.dot(x_ref[...], w_ref[...],
                              preferred_element_type=jnp.float32)
                  + b_ref[:1, :])


def _dense(x, w, b, interpret=False):
    n, fi = x.shape
    fo = w.shape[1]
    npad = ((n + _RT - 1) // _RT) * _RT
    xp = jnp.pad(x, ((0, npad - n), (0, 0)))
    b2 = jnp.broadcast_to(b[None, :], (8, fo))
    out = pl.pallas_call(
        _matmul_kernel,
        grid=(npad // _RT,),
        in_specs=[
            pl.BlockSpec((_RT, fi), lambda i: (i, 0)),
            pl.BlockSpec((fi, fo), lambda i: (0, 0)),
            pl.BlockSpec((8, fo), lambda i: (0, 0)),
        ],
        out_specs=pl.BlockSpec((_RT, fo), lambda i: (i, 0)),
        out_shape=jax.ShapeDtypeStruct((npad, fo), jnp.float32),
        compiler_params=pltpu.CompilerParams(
            dimension_semantics=("arbitrary",)),
        interpret=interpret,
    )(xp, w, b2)
    return out[:n]


def _batchnorm_j(h, g, b):
    mu = jnp.mean(h, axis=0)
    var = jnp.var(h, axis=0)
    return (h - mu) / jnp.sqrt(var + _EPS) * g + b


def _graph_norm_j(h, batch, w, b, ms):
    ones = jnp.ones((h.shape[0], 1), dtype=h.dtype)
    cnt = jnp.maximum(jax.ops.segment_sum(ones, batch,
                                          num_segments=_N_GRAPHS), 1.0)
    mean = jax.ops.segment_sum(h, batch, num_segments=_N_GRAPHS) / cnt
    out = h - mean[batch] * ms
    var = jax.ops.segment_sum(out * out, batch, num_segments=_N_GRAPHS) / cnt
    std = jnp.sqrt(var[batch] + _EPS)
    return w * out / std + b


def kernel(x, edge_index, batch, params):
    del edge_index
    h = _dense(x, params['embed_W'], params['embed_b'])
    h = _batchnorm_j(h, params['bn0_g'], params['bn0_b'])
    for lp in params['layers']:
        x_in = h
        s = _dense(h, lp['Ws'], lp['bs'])
        feat = _dense(h, lp['Wh'], lp['bh'])
        mean_a, max_a = _knn_aggregate(s, feat, batch, _N_GRAPHS, _K_NN)
        h2 = jnp.concatenate([h, mean_a, max_a], axis=1)
        h2 = _dense(h2, lp['Wo'], lp['bo'])
        h2 = _graph_norm_j(h2, batch, lp['gn_w'], lp['gn_b'], lp['gn_ms'])
        h2 = jax.nn.relu(h2)
        h = h2 + x_in
    ones = jnp.ones((h.shape[0], 1), dtype=h.dtype)
    cnt = jnp.maximum(jax.ops.segment_sum(ones, batch,
                                          num_segments=_N_GRAPHS), 1.0)
    pooled = jax.ops.segment_sum(h, batch, num_segments=_N_GRAPHS) / cnt
    c = params['cls']
    cls = jax.nn.relu(pooled @ c['W1'] + c['b1'])
    cls = jax.nn.relu(cls @ c['W2'] + c['b2'])
    cls = cls @ c['W3'] + c['b3']
    r = params['reg']
    reg = jax.nn.relu(pooled @ r['W1'] + r['b1'])
    reg = _batchnorm_j(reg, r['bn1_g'], r['bn1_b'])
    reg = jax.nn.relu(reg @ r['W2'] + r['b2'])
    reg = _batchnorm_j(reg, r['bn2_g'], r['bn2_b'])
    reg = reg @ r['W3'] + r['b3']
    return (cls, reg)


# speedup vs baseline: 20.1293x; 1.6008x over previous
import functools

import jax
import jax.numpy as jnp
from jax.experimental import pallas as pl
from jax.experimental.pallas import tpu as pltpu

_N_GRAPHS = 64
_K_NN = 32
_EPS = 1e-5
_RT = 256
_CT = 256

_SMEM = getattr(pltpu, "SMEM", None)
if _SMEM is None:
    _SMEM = pltpu.TPUMemorySpace.SMEM


def _knn_kernel(bounds_ref, s_r_ref, s_c_ref, b_r_ref, b_c_ref, ft_ref,
                ft_t_ref, mean_ref, max_ref, *, s_dim, ct, k_nn):
    i = pl.program_id(0)
    c0 = bounds_ref[i, 0]
    c1 = bounds_ref[i, 1]
    sr = s_r_ref[...]
    br = b_r_ref[...][:, :1]
    rt = sr.shape[0]
    nf = ft_ref.shape[1]

    sr2 = sr * sr
    sq_r = sr2[:, 0:1]
    for dd in range(1, s_dim):
        sq_r = sq_r + sr2[:, dd:dd + 1]

    def d2_same(c):
        sc = s_c_ref[:, pl.ds(c * ct, ct)]
        dot = jax.lax.dot(sr, sc, preferred_element_type=jnp.float32)
        sc2 = sc * sc
        sq_c = sc2[0:1, :]
        for dd in range(1, s_dim):
            sq_c = sq_c + sc2[dd:dd + 1, :]
        d2 = sq_r + sq_c - 2.0 * dot
        bc = b_c_ref[:1, pl.ds(c * ct, ct)]
        return d2, br == bc

    def tile_step(c, best):
        d2, same = d2_same(c)
        cand = jnp.concatenate(
            [best, jnp.where(same, d2, jnp.inf)], axis=1)
        tk = jnp.full((rt, 1), -jnp.inf, jnp.float32)
        outs = []
        for _ in range(k_nn):
            cur = jnp.where(cand > tk, cand, jnp.inf)
            tk = jnp.min(cur, axis=1, keepdims=True)
            outs.append(tk)
        return jnp.concatenate(outs, axis=1)

    best = jax.lax.fori_loop(c0, c1, tile_step,
                             jnp.full((rt, k_nn), jnp.inf, jnp.float32))
    t = best[:, k_nn - 1:k_nn]

    def agg_step(c, carry):
        msum, mmax, cnt = carry
        d2, same = d2_same(c)
        sel = same & (d2 <= t)
        sc = s_c_ref[:, pl.ds(c * ct, ct)]
        d = jnp.zeros((rt, ct), jnp.float32)
        for dd in range(s_dim):
            diff = sr[:, dd:dd + 1] - sc[dd:dd + 1, :]
            d = d + diff * diff
        w = jnp.where(sel, jnp.exp(-10.0 * d), 0.0)
        ftc = ft_ref[pl.ds(c * ct, ct), :]
        msum = msum + jax.lax.dot(w, ftc, preferred_element_type=jnp.float32)
        cnt = cnt + jnp.sum(sel.astype(jnp.float32), axis=1, keepdims=True)
        nf = ft_ref.shape[1]
        cols = []
        for d in range(nf):
            fr = ft_t_ref[d:d + 1, pl.ds(c * ct, ct)]
            m2 = jnp.where(sel, w * fr, -jnp.inf)
            cols.append(jnp.max(m2, axis=1, keepdims=True))
        mmax = jnp.maximum(mmax, jnp.concatenate(cols, axis=1))
        return msum, mmax, cnt

    init = (jnp.zeros((rt, nf), jnp.float32),
            jnp.full((rt, nf), -jnp.inf, jnp.float32),
            jnp.zeros((rt, 1), jnp.float32))
    msum, mmax, cnt = jax.lax.fori_loop(c0, c1, agg_step, init)
    mean_ref[...] = msum / jnp.maximum(cnt, 1.0)
    max_ref[...] = jnp.where(jnp.isfinite(mmax), mmax, 0.0)


def _knn_aggregate(s, feat, batch, n_graphs, k_nn, interpret=False):
    n, s_dim = s.shape
    nf = feat.shape[1]
    npad = ((n + _RT - 1) // _RT) * _RT
    nt = npad // _RT
    pad = npad - n

    s_r = jnp.pad(s, ((0, pad), (0, 8 - s_dim)))
    s_c = s_r.T.reshape(8, npad)
    b_r = jnp.pad(batch.astype(jnp.float32), (0, pad),
                  constant_values=float(n_graphs + 63))
    b_c = jnp.pad(batch.astype(jnp.float32), (0, pad),
                  constant_values=float(n_graphs + 191))
    b_r = jnp.broadcast_to(b_r[:, None], (npad, 8))
    b_c = jnp.broadcast_to(b_c[None, :], (8, npad))
    ft = jnp.pad(feat, ((0, pad), (0, 0)))
    ft_t = ft.T.reshape(nf, npad)

    gids = jnp.arange(n_graphs, dtype=batch.dtype)
    g_start = jnp.searchsorted(batch, gids, side="left").astype(jnp.int32)
    g_end = jnp.searchsorted(batch, gids, side="right").astype(jnp.int32)
    first = jnp.minimum(jnp.arange(nt) * _RT, n - 1)
    last = jnp.minimum(jnp.arange(nt) * _RT + _RT - 1, n - 1)
    lo = g_start[batch[first]]
    hi = g_end[batch[last]]
    bounds = jnp.stack([lo // _CT, (hi + _CT - 1) // _CT], axis=1)

    kern = functools.partial(_knn_kernel, s_dim=s_dim, ct=_CT, k_nn=k_nn)
    mean_a, max_a = pl.pallas_call(
        kern,
        grid=(nt,),
        in_specs=[
            pl.BlockSpec(memory_space=_SMEM),
            pl.BlockSpec((_RT, 8), lambda i: (i, 0)),
            pl.BlockSpec((8, npad), lambda i: (0, 0)),
            pl.BlockSpec((_RT, 8), lambda i: (i, 0)),
            pl.BlockSpec((8, npad), lambda i: (0, 0)),
            pl.BlockSpec((npad, nf), lambda i: (0, 0)),
            pl.BlockSpec((nf, npad), lambda i: (0, 0)),
        ],
        out_specs=[
            pl.BlockSpec((_RT, nf), lambda i: (i, 0)),
            pl.BlockSpec((_RT, nf), lambda i: (i, 0)),
        ],
        out_shape=[
            jax.ShapeDtypeStruct((npad, nf), jnp.float32),
            jax.ShapeDtypeStruct((npad, nf), jnp.float32),
        ],
        compiler_params=pltpu.CompilerParams(
            dimension_semantics=("arbitrary",)),
        interpret=interpret,
    )(bounds, s_r, s_c, b_r, b_c, ft, ft_t)
    return mean_a[:n], max_a[:n]


def _matmul_kernel(x_ref, w_ref, b_ref, o_ref):
    o_ref[...] = (jax.lax.dot(x_ref[...], w_ref[...],
                              preferred_element_type=jnp.float32)
                  + b_ref[:1, :])


def _dense(x, w, b, interpret=False):
    n, fi = x.shape
    fo = w.shape[1]
    npad = ((n + _RT - 1) // _RT) * _RT
    xp = jnp.pad(x, ((0, npad - n), (0, 0)))
    b2 = jnp.broadcast_to(b[None, :], (8, fo))
    out = pl.pallas_call(
        _matmul_kernel,
        grid=(npad // _RT,),
        in_specs=[
            pl.BlockSpec((_RT, fi), lambda i: (i, 0)),
            pl.BlockSpec((fi, fo), lambda i: (0, 0)),
            pl.BlockSpec((8, fo), lambda i: (0, 0)),
        ],
        out_specs=pl.BlockSpec((_RT, fo), lambda i: (i, 0)),
        out_shape=jax.ShapeDtypeStruct((npad, fo), jnp.float32),
        compiler_params=pltpu.CompilerParams(
            dimension_semantics=("arbitrary",)),
        interpret=interpret,
    )(xp, w, b2)
    return out[:n]


def _batchnorm_j(h, g, b):
    mu = jnp.mean(h, axis=0)
    var = jnp.var(h, axis=0)
    return (h - mu) / jnp.sqrt(var + _EPS) * g + b


def _graph_norm_j(h, batch, w, b, ms):
    ones = jnp.ones((h.shape[0], 1), dtype=h.dtype)
    cnt = jnp.maximum(jax.ops.segment_sum(ones, batch,
                                          num_segments=_N_GRAPHS), 1.0)
    mean = jax.ops.segment_sum(h, batch, num_segments=_N_GRAPHS) / cnt
    out = h - mean[batch] * ms
    var = jax.ops.segment_sum(out * out, batch, num_segments=_N_GRAPHS) / cnt
    std = jnp.sqrt(var[batch] + _EPS)
    return w * out / std + b


def kernel(x, edge_index, batch, params):
    del edge_index
    h = _dense(x, params['embed_W'], params['embed_b'])
    h = _batchnorm_j(h, params['bn0_g'], params['bn0_b'])
    for lp in params['layers']:
        x_in = h
        s = _dense(h, lp['Ws'], lp['bs'])
        feat = _dense(h, lp['Wh'], lp['bh'])
        mean_a, max_a = _knn_aggregate(s, feat, batch, _N_GRAPHS, _K_NN)
        h2 = jnp.concatenate([h, mean_a, max_a], axis=1)
        h2 = _dense(h2, lp['Wo'], lp['bo'])
        h2 = _graph_norm_j(h2, batch, lp['gn_w'], lp['gn_b'], lp['gn_ms'])
        h2 = jax.nn.relu(h2)
        h = h2 + x_in
    ones = jnp.ones((h.shape[0], 1), dtype=h.dtype)
    cnt = jnp.maximum(jax.ops.segment_sum(ones, batch,
                                          num_segments=_N_GRAPHS), 1.0)
    pooled = jax.ops.segment_sum(h, batch, num_segments=_N_GRAPHS) / cnt
    c = params['cls']
    cls = jax.nn.relu(pooled @ c['W1'] + c['b1'])
    cls = jax.nn.relu(cls @ c['W2'] + c['b2'])
    cls = cls @ c['W3'] + c['b3']
    r = params['reg']
    reg = jax.nn.relu(pooled @ r['W1'] + r['b1'])
    reg = _batchnorm_j(reg, r['bn1_g'], r['bn1_b'])
    reg = jax.nn.relu(reg @ r['W2'] + r['b2'])
    reg = _batchnorm_j(reg, r['bn2_g'], r['bn2_b'])
    reg = reg @ r['W3'] + r['b3']
    return (cls, reg)
